# baseline (speedup 1.0000x reference)
import functools

import jax
import jax.numpy as jnp
from jax import lax
from jax.experimental import pallas as pl
from jax.experimental.pallas import tpu as pltpu

_LOG2 = 0.6931471805599453
_NI = 3
_GAMMA = 10.0


def _ssp(x):
    return jnp.maximum(x, 0.0) + jnp.log1p(jnp.exp(-jnp.abs(x))) - _LOG2


def _schnet_kernel(z_ref, r_ref, emb_ref, wf1_ref, bf1_ref, wf2_ref, bf2_ref,
                   win_ref, bin_ref, wo1_ref, bo1_ref, wo2_ref, bo2_ref,
                   wa1_ref, ba1_ref, wa2_ref, out_ref, *, A, G, NF, F, NC):
    f32 = jnp.float32

    zrow = z_ref[0]
    ohT = (lax.broadcasted_iota(jnp.int32, (NC, A), 0) == zrow).astype(f32)
    x = lax.dot_general(ohT, emb_ref[...],
                        (((0,), (0,)), ((), ())),
                        preferred_element_type=f32)

    rb = r_ref[0]
    centers = (lax.broadcasted_iota(jnp.int32, (1, 1, G), 2).astype(f32)
               * (1.0 / (G - 1)))
    d = rb[:, :, None] - centers
    rbf = jnp.exp((-_GAMMA) * d * d)
    rbf2 = rbf.reshape(A * A, G)

    h = _ssp(jnp.dot(rbf2, wf1_ref[...], preferred_element_type=f32)
             + bf1_ref[...])
    wf = _ssp(jnp.dot(h, wf2_ref[...], preferred_element_type=f32)
              + bf2_ref[...])
    wf3 = wf.reshape(A, A, NF)

    for _ in range(_NI):
        xf = jnp.dot(x, win_ref[...], preferred_element_type=f32) + bin_ref[...]
        y = jnp.sum(wf3 * xf[None, :, :], axis=1)
        v = _ssp(jnp.dot(y, wo1_ref[...], preferred_element_type=f32)
                 + bo1_ref[...])
        v = jnp.dot(v, wo2_ref[...], preferred_element_type=f32) + bo2_ref[...]
        x = x + v

    xa = _ssp(jnp.dot(x, wa1_ref[...], preferred_element_type=f32)
              + ba1_ref[...])
    o = lax.dot_general(wa2_ref[...], xa,
                        (((1,), (1,)), ((), ())),
                        preferred_element_type=f32)
    out_ref[0] = o


def kernel(z, r, emb, W_f1, b_f1, W_f2, b_f2, W_in, b_in, W_o1, b_o1,
           W_o2, b_o2, W_a1, b_a1, W_a2, b_a2):
    B, A = z.shape
    G, NF = W_f1.shape
    F = emb.shape[1]
    NC = 128

    z3 = z.astype(jnp.int32).reshape(B, 1, A)
    emb_pad = jnp.zeros((NC, F), jnp.float32).at[:emb.shape[0]].set(emb)
    row = lambda b: b.reshape(1, -1).astype(jnp.float32)

    full = lambda shape: pl.BlockSpec(shape, lambda b: (0,) * len(shape))

    out = pl.pallas_call(
        functools.partial(_schnet_kernel, A=A, G=G, NF=NF, F=F, NC=NC),
        grid=(B,),
        in_specs=[
            pl.BlockSpec((1, 1, A), lambda b: (b, 0, 0)),
            pl.BlockSpec((1, A, A), lambda b: (b, 0, 0)),
            full((NC, F)),
            full((G, NF)), full((1, NF)),
            full((NF, NF)), full((1, NF)),
            full((F, NF)), full((1, NF)),
            full((NF, F)), full((1, F)),
            full((F, F)), full((1, F)),
            full((F, F)), full((1, F)),
            full((1, F)),
        ],
        out_specs=pl.BlockSpec((1, 1, A), lambda b: (b, 0, 0)),
        out_shape=jax.ShapeDtypeStruct((B, 1, A), jnp.float32),
        compiler_params=pltpu.CompilerParams(
            dimension_semantics=("parallel",)),
    )(z3, r, emb_pad, W_f1, row(b_f1), W_f2, row(b_f2), W_in, row(b_in),
      W_o1, row(b_o1), W_o2, row(b_o2), W_a1, row(b_a1), W_a2.reshape(1, F))

    return out.reshape(B, A, 1) + b_a2[0]


# speedup vs baseline: 1.5944x; 1.5944x over previous
import functools

import jax
import jax.numpy as jnp
from jax import lax
from jax.experimental import pallas as pl
from jax.experimental.pallas import tpu as pltpu

_LOG2 = 0.6931471805599453
_NI = 3
_GAMMA = 10.0
_MB = 2


def _ssp_fast(x):
    return jnp.log(1.0 + jnp.exp(x)) - _LOG2


def _ssp(x):
    return jnp.maximum(x, 0.0) + jnp.log1p(jnp.exp(-jnp.abs(x))) - _LOG2


def _schnet_kernel(z_ref, r_ref, emb_ref, wf1_ref, bf1_ref, wf2_ref, bf2_ref,
                   win_ref, bin_ref, wo1_ref, bo1_ref, wo2_ref, bo2_ref,
                   wa1_ref, ba1_ref, wa2_ref, out_ref, *, MB, A, G, NF, F, NC):
    f32 = jnp.float32
    M = MB * A

    zrow = z_ref[0]
    ohT = (lax.broadcasted_iota(jnp.int32, (NC, M), 0) == zrow).astype(f32)
    x = lax.dot_general(ohT, emb_ref[...],
                        (((0,), (0,)), ((), ())),
                        preferred_element_type=f32)

    rb = r_ref[...]
    centers = (lax.broadcasted_iota(jnp.int32, (1, 1, G), 2).astype(f32)
               * (1.0 / (G - 1)))
    d = rb[:, :, None] - centers
    rbf = jnp.exp((-_GAMMA) * d * d)
    rbf2 = rbf.reshape(M * A, G)

    h = _ssp_fast(jnp.dot(rbf2, wf1_ref[...], preferred_element_type=f32)
                  + bf1_ref[...])
    wf = _ssp_fast(jnp.dot(h, wf2_ref[...], preferred_element_type=f32)
                   + bf2_ref[...])
    wf4 = wf.reshape(MB, A, A, NF)

    for _ in range(_NI):
        xf = jnp.dot(x, win_ref[...], preferred_element_type=f32) + bin_ref[...]
        xf4 = xf.reshape(MB, 1, A, NF)
        y = jnp.sum(wf4 * xf4, axis=2)
        y = y.reshape(M, NF)
        v = _ssp(jnp.dot(y, wo1_ref[...], preferred_element_type=f32)
                 + bo1_ref[...])
        v = jnp.dot(v, wo2_ref[...], preferred_element_type=f32) + bo2_ref[...]
        x = x + v

    xa = _ssp(jnp.dot(x, wa1_ref[...], preferred_element_type=f32)
              + ba1_ref[...])
    o = lax.dot_general(wa2_ref[...], xa,
                        (((1,), (1,)), ((), ())),
                        preferred_element_type=f32)
    out_ref[0] = o


def kernel(z, r, emb, W_f1, b_f1, W_f2, b_f2, W_in, b_in, W_o1, b_o1,
           W_o2, b_o2, W_a1, b_a1, W_a2, b_a2):
    B, A = z.shape
    G, NF = W_f1.shape
    F = emb.shape[1]
    NC = 128
    MB = _MB
    M = MB * A

    z3 = z.astype(jnp.int32).reshape(B // MB, 1, M)
    r2 = r.reshape(B * A, A)
    emb_pad = jnp.zeros((NC, F), jnp.float32).at[:emb.shape[0]].set(emb)
    row = lambda b: b.reshape(1, -1).astype(jnp.float32)

    full = lambda shape: pl.BlockSpec(shape, lambda b: (0,) * len(shape))

    out = pl.pallas_call(
        functools.partial(_schnet_kernel, MB=MB, A=A, G=G, NF=NF, F=F, NC=NC),
        grid=(B // MB,),
        in_specs=[
            pl.BlockSpec((1, 1, M), lambda b: (b, 0, 0)),
            pl.BlockSpec((M, A), lambda b: (b, 0)),
            full((NC, F)),
            full((G, NF)), full((1, NF)),
            full((NF, NF)), full((1, NF)),
            full((F, NF)), full((1, NF)),
            full((NF, F)), full((1, F)),
            full((F, F)), full((1, F)),
            full((F, F)), full((1, F)),
            full((1, F)),
        ],
        out_specs=pl.BlockSpec((1, 1, M), lambda b: (b, 0, 0)),
        out_shape=jax.ShapeDtypeStruct((B // MB, 1, M), jnp.float32),
        compiler_params=pltpu.CompilerParams(
            dimension_semantics=("parallel",)),
    )(z3, r2, emb_pad, W_f1, row(b_f1), W_f2, row(b_f2), W_in, row(b_in),
      W_o1, row(b_o1), W_o2, row(b_o2), W_a1, row(b_a1), W_a2.reshape(1, F))

    return out.reshape(B, A, 1) + b_a2[0]
